# baseline (speedup 1.0000x reference)
import functools

import jax
import jax.numpy as jnp
from jax import lax
from jax.experimental import pallas as pl
from jax.experimental.pallas import tpu as pltpu
from jax.experimental.pallas import tpu_sc as plsc

_SEQ_LEN = 8192
_OUT_DIM = 1024
_NC = 2
_NS = 16
_NW = _NC * _NS
_ROWS_PER_W = _SEQ_LEN // _NW


_CHUNK = 32
_NCHUNKS = _ROWS_PER_W // _CHUNK
_NBUF = 2


def _make_sc_copy():
    mesh = plsc.VectorSubcoreMesh(core_axis_name="c", subcore_axis_name="s")

    @functools.partial(
        pl.kernel,
        mesh=mesh,
        out_type=jax.ShapeDtypeStruct((_SEQ_LEN, _OUT_DIM), jnp.float32),
        scratch_types=[
            pltpu.VMEM_SHARED((_NS, _NBUF, _CHUNK, _OUT_DIM), jnp.float32),
            pltpu.SemaphoreType.DMA,
            pltpu.SemaphoreType.DMA,
        ],
    )
    def copy_k(table_hbm, out_hbm, sbuf, gsem, ssem):
        sid = lax.axis_index("s")
        wid = sid * _NC + lax.axis_index("c")
        base = wid * _ROWS_PER_W
        buf = sbuf.at[sid]

        def gather(i):
            return pltpu.async_copy(
                table_hbm.at[pl.ds(base + i * _CHUNK, _CHUNK)],
                buf.at[i % _NBUF],
                gsem,
            )

        def scatter(i):
            return pltpu.async_copy(
                buf.at[i % _NBUF],
                out_hbm.at[pl.ds(base + i * _CHUNK, _CHUNK)],
                ssem,
            )

        gathers = [None] * _NCHUNKS
        scatters = [None] * _NCHUNKS
        for i in range(_NCHUNKS + 1):
            if i < _NCHUNKS:
                if i >= _NBUF:
                    scatters[i - _NBUF].wait()
                gathers[i] = gather(i)
            if i >= 1:
                gathers[i - 1].wait()
                scatters[i - 1] = scatter(i - 1)
        for i in range(max(0, _NCHUNKS - _NBUF), _NCHUNKS):
            scatters[i].wait()

    return copy_k


_sc_copy = _make_sc_copy()


@jax.jit
def kernel(inputs, pos_table):
    del inputs
    return _sc_copy(pos_table)


# speedup vs baseline: 1.0458x; 1.0458x over previous
import functools

import jax
import jax.numpy as jnp
from jax import lax
from jax.experimental import pallas as pl
from jax.experimental.pallas import tpu as pltpu
from jax.experimental.pallas import tpu_sc as plsc

_SEQ_LEN = 8192
_OUT_DIM = 1024
_NC = 2
_NS = 16
_NW = _NC * _NS
_ROWS_PER_W = _SEQ_LEN // _NW


_BUF_ROWS = 56
_CHUNKS = [56, 56, 56, 56, 32]
_OFFS = [0, 56, 112, 168, 224]
_NCHUNKS = len(_CHUNKS)
_NBUF = 2


def _make_sc_copy():
    mesh = plsc.VectorSubcoreMesh(core_axis_name="c", subcore_axis_name="s")

    @functools.partial(
        pl.kernel,
        mesh=mesh,
        out_type=jax.ShapeDtypeStruct((_SEQ_LEN, _OUT_DIM), jnp.float32),
        scratch_types=[
            pltpu.VMEM((_NBUF, _BUF_ROWS, _OUT_DIM), jnp.float32),
            pltpu.SemaphoreType.DMA,
            pltpu.SemaphoreType.DMA,
        ],
    )
    def copy_k(table_hbm, out_hbm, buf, gsem, ssem):
        wid = lax.axis_index("s") * _NC + lax.axis_index("c")
        base = wid * _ROWS_PER_W

        def gather(i):
            return pltpu.async_copy(
                table_hbm.at[pl.ds(base + _OFFS[i], _CHUNKS[i])],
                buf.at[i % _NBUF, pl.ds(0, _CHUNKS[i])],
                gsem,
            )

        def scatter(i):
            return pltpu.async_copy(
                buf.at[i % _NBUF, pl.ds(0, _CHUNKS[i])],
                out_hbm.at[pl.ds(base + _OFFS[i], _CHUNKS[i])],
                ssem,
            )

        gathers = [None] * _NCHUNKS
        scatters = [None] * _NCHUNKS
        for i in range(_NCHUNKS + 1):
            if i < _NCHUNKS:
                if i >= _NBUF:
                    scatters[i - _NBUF].wait()
                gathers[i] = gather(i)
            if i >= 1:
                gathers[i - 1].wait()
                scatters[i - 1] = scatter(i - 1)
        for i in range(max(0, _NCHUNKS - _NBUF), _NCHUNKS):
            scatters[i].wait()

    return copy_k


_sc_copy = _make_sc_copy()


@jax.jit
def kernel(inputs, pos_table):
    del inputs
    return _sc_copy(pos_table)
